# baseline (speedup 1.0000x reference)
import jax
import jax.numpy as jnp
from jax import lax
from jax.experimental import pallas as pl
from jax.experimental.pallas import tpu as pltpu

B, N, M, D = 8, 2048, 2048, 64
K = 128


def _prep(x_ref, y_ref, xa_ref, ya_ref):
    f32 = jnp.float32
    bf16 = jnp.bfloat16
    xv = x_ref[0]
    yv = y_ref[0]
    x2 = jnp.sum(xv * xv, axis=1, keepdims=True)
    y2 = jnp.sum(yv * yv, axis=1, keepdims=True)
    x2_hi = x2.astype(bf16)
    x2_lo = (x2 - x2_hi.astype(f32)).astype(bf16)
    y2_hi = y2.astype(bf16)
    y2_lo = (y2 - y2_hi.astype(f32)).astype(bf16)
    one_col = jnp.ones((N, 2), bf16)
    zpad = jnp.zeros((N, K - D - 4), bf16)
    xa_ref[...] = jnp.concatenate(
        [(-2.0 * xv).astype(bf16), x2_hi, x2_lo, one_col, zpad], axis=1)
    ya_ref[...] = jnp.concatenate(
        [yv.astype(bf16), one_col, y2_hi, y2_lo, zpad], axis=1)


def _compute(xa_ref, ya_ref, acc_ref):
    d = lax.dot_general(xa_ref[...], ya_ref[...],
                        (((1,), (1,)), ((), ())),
                        preferred_element_type=jnp.float32)

    pm = d[:, 0:128]
    for k in range(1, M // 128):
        pm = jnp.minimum(pm, d[:, k * 128:(k + 1) * 128])
    rm = jnp.min(pm, axis=1)

    cm = jnp.min(d, axis=0)

    acc_ref[0, 0] += (
        jnp.sum(jnp.maximum(cm, 0.0)) * (1.0 / (M * B))
        + jnp.sum(jnp.maximum(rm, 0.0)) * (1.0 / (N * B)))


def _chamfer_kernel(x_ref, y_ref, acc_ref, xa0, ya0, xa1, ya1):
    s = pl.program_id(0)

    @pl.when(s == 0)
    def _():
        acc_ref[0, 0] = 0.0

    @pl.when(s % 2 == 0)
    def _():
        _prep(x_ref, y_ref, xa0, ya0)

    @pl.when(s % 2 == 1)
    def _():
        _prep(x_ref, y_ref, xa1, ya1)

    @pl.when(s % 2 == 1)
    def _():
        _compute(xa0, ya0, acc_ref)

    @pl.when((s > 0) & (s % 2 == 0))
    def _():
        _compute(xa1, ya1, acc_ref)


@jax.jit
def kernel(x, y):
    acc = pl.pallas_call(
        _chamfer_kernel,
        grid=(B + 1,),
        in_specs=[
            pl.BlockSpec((1, N, D), lambda s: (jnp.minimum(s, B - 1), 0, 0)),
            pl.BlockSpec((1, M, D), lambda s: (jnp.minimum(s, B - 1), 0, 0)),
        ],
        out_specs=pl.BlockSpec(
            (1, 1), lambda s: (0, 0), memory_space=pltpu.SMEM),
        out_shape=jax.ShapeDtypeStruct((1, 1), jnp.float32),
        scratch_shapes=[
            pltpu.VMEM((N, K), jnp.bfloat16),
            pltpu.VMEM((M, K), jnp.bfloat16),
            pltpu.VMEM((N, K), jnp.bfloat16),
            pltpu.VMEM((M, K), jnp.bfloat16),
        ],
    )(x, y)
    return acc[0, 0]


# speedup vs baseline: 1.1165x; 1.1165x over previous
import jax
import jax.numpy as jnp
from jax import lax
from jax.experimental import pallas as pl
from jax.experimental.pallas import tpu as pltpu

B, N, M, D = 8, 2048, 2048, 64
K = 128


def _one_batch(xv, yv):
    f32 = jnp.float32
    bf16 = jnp.bfloat16
    x2 = jnp.sum(xv * xv, axis=1, keepdims=True)
    y2 = jnp.sum(yv * yv, axis=1, keepdims=True)
    x2_hi = x2.astype(bf16)
    x2_lo = (x2 - x2_hi.astype(f32)).astype(bf16)
    y2_hi = y2.astype(bf16)
    y2_lo = (y2 - y2_hi.astype(f32)).astype(bf16)
    one_col = jnp.ones((N, 2), bf16)
    zpad = jnp.zeros((N, K - D - 4), bf16)
    xa = jnp.concatenate(
        [(-2.0 * xv).astype(bf16), x2_hi, x2_lo, one_col, zpad], axis=1)
    ya = jnp.concatenate(
        [yv.astype(bf16), one_col, y2_hi, y2_lo, zpad], axis=1)

    d = lax.dot_general(xa, ya, (((1,), (1,)), ((), ())),
                        preferred_element_type=f32)

    pm = d[:, 0:128]
    for k in range(1, M // 128):
        pm = jnp.minimum(pm, d[:, k * 128:(k + 1) * 128])
    rm = jnp.min(pm, axis=1)

    cm = jnp.min(d, axis=0)

    return (jnp.sum(jnp.maximum(cm, 0.0)) * (1.0 / (M * B))
            + jnp.sum(jnp.maximum(rm, 0.0)) * (1.0 / (N * B)))


def _chamfer_kernel(x_ref, y_ref, acc_ref):
    s = pl.program_id(0)

    @pl.when(s == 0)
    def _():
        acc_ref[0, 0] = 0.0

    acc_ref[0, 0] += _one_batch(x_ref[0], y_ref[0]) + _one_batch(
        x_ref[1], y_ref[1])


@jax.jit
def kernel(x, y):
    acc = pl.pallas_call(
        _chamfer_kernel,
        grid=(B // 2,),
        in_specs=[
            pl.BlockSpec((2, N, D), lambda s: (s, 0, 0)),
            pl.BlockSpec((2, M, D), lambda s: (s, 0, 0)),
        ],
        out_specs=pl.BlockSpec(
            (1, 1), lambda s: (0, 0), memory_space=pltpu.SMEM),
        out_shape=jax.ShapeDtypeStruct((1, 1), jnp.float32),
    )(x, y)
    return acc[0, 0]
